# Initial kernel scaffold; baseline (speedup 1.0000x reference)
import jax
import jax.numpy as jnp
from jax.experimental import pallas as pl


def kernel(char_bytes, lut):
    raise NotImplementedError("write your pallas kernel here")


# speedup vs baseline: 304.4192x; 304.4192x over previous
import functools

import jax
import jax.numpy as jnp
from jax import lax
from jax.experimental import pallas as pl
from jax.experimental.pallas import tpu as pltpu
from jax.experimental.pallas import tpu_sc as plsc

_NW = 32
_LANES = 16
_CHUNK = 12800


@functools.partial(jax.jit, static_argnums=0)
def _lut_gather(n_total, codes_flat, lut32):
    per_w = n_total // _NW
    n_chunks = per_w // _CHUNK
    mesh = plsc.VectorSubcoreMesh(core_axis_name="c", subcore_axis_name="s")

    @functools.partial(
        pl.kernel,
        out_type=jax.ShapeDtypeStruct((n_total,), jnp.int32),
        mesh=mesh,
        compiler_params=pltpu.CompilerParams(needs_layout_passes=False),
        scratch_types=[
            pltpu.VMEM((256,), jnp.int32),
            pltpu.VMEM((_CHUNK,), jnp.int32),
            pltpu.VMEM((_CHUNK,), jnp.int32),
            pltpu.VMEM((_CHUNK,), jnp.int32),
            pltpu.VMEM((_CHUNK,), jnp.int32),
            pltpu.SemaphoreType.DMA,
            pltpu.SemaphoreType.DMA,
            pltpu.SemaphoreType.DMA,
            pltpu.SemaphoreType.DMA,
        ],
    )
    def k(codes_hbm, lut_hbm, out_hbm, lut_v, in_v0, in_v1, out_v0, out_v1,
          isem0, isem1, osem0, osem1):
        wid = lax.axis_index("s") * 2 + lax.axis_index("c")
        base = wid * per_w
        pltpu.sync_copy(lut_hbm, lut_v)
        in_bufs = (in_v0, in_v1)
        out_bufs = (out_v0, out_v1)
        isems = (isem0, isem1)
        osems = (osem0, osem1)
        in_cps = [None, None]
        out_cps = [None, None]

        def start_in(g):
            b = g % 2
            in_cps[b] = pltpu.async_copy(
                codes_hbm.at[pl.ds(base + g * _CHUNK, _CHUNK)],
                in_bufs[b], isems[b])

        start_in(0)
        for g in range(n_chunks):
            b = g % 2
            if g + 1 < n_chunks:
                start_in(g + 1)
            in_cps[b].wait()
            if out_cps[b] is not None:
                out_cps[b].wait()
            in_v, out_v = in_bufs[b], out_bufs[b]

            @plsc.parallel_loop(0, _CHUNK, _LANES, unroll=8)
            def body(i):
                idx = in_v[pl.ds(i, _LANES)]
                out_v[pl.ds(i, _LANES)] = plsc.load_gather(lut_v, [idx])

            out_cps[b] = pltpu.async_copy(
                out_bufs[b],
                out_hbm.at[pl.ds(base + g * _CHUNK, _CHUNK)], osems[b])

        for b in range(2):
            if out_cps[b] is not None:
                out_cps[b].wait()

    return k(codes_flat, lut32)


def kernel(char_bytes, lut):
    B, L = char_bytes.shape
    n = B * L
    lut32 = lut.astype(jnp.int32)
    flat = char_bytes.reshape(n).astype(jnp.int32)
    grain = _NW * _CHUNK
    n_pad = -(-n // grain) * grain
    if n_pad != n:
        flat = jnp.pad(flat, (0, n_pad - n))
    out = _lut_gather(n_pad, flat, lut32)
    return out[:n].reshape(B, L).astype(lut.dtype)


# speedup vs baseline: 525.6257x; 1.7267x over previous
import functools

import jax
import jax.numpy as jnp
from jax import lax
from jax.experimental import pallas as pl
from jax.experimental.pallas import tpu as pltpu
from jax.experimental.pallas import tpu_sc as plsc

_NW = 32
_LANES = 16
_ROWS_PER_CHUNK = 64


@functools.partial(jax.jit, static_argnums=(0, 1))
def _lut_gather(n_rows, n_cols, codes, lut32):
    rows_per_w = n_rows // _NW
    n_chunks = rows_per_w // _ROWS_PER_CHUNK
    offs = list(range(0, n_cols - _LANES + 1, _LANES))
    if offs[-1] + _LANES < n_cols:
        offs.append(n_cols - _LANES)
    mesh = plsc.VectorSubcoreMesh(core_axis_name="c", subcore_axis_name="s")

    @functools.partial(
        pl.kernel,
        out_type=jax.ShapeDtypeStruct((n_rows, n_cols), jnp.int32),
        mesh=mesh,
        compiler_params=pltpu.CompilerParams(needs_layout_passes=False),
        scratch_types=[
            pltpu.VMEM((256,), jnp.int32),
            pltpu.VMEM((_ROWS_PER_CHUNK, n_cols), jnp.int32),
            pltpu.VMEM((_ROWS_PER_CHUNK, n_cols), jnp.int32),
            pltpu.VMEM((_ROWS_PER_CHUNK, n_cols), jnp.int32),
            pltpu.VMEM((_ROWS_PER_CHUNK, n_cols), jnp.int32),
            pltpu.SemaphoreType.DMA,
            pltpu.SemaphoreType.DMA,
            pltpu.SemaphoreType.DMA,
            pltpu.SemaphoreType.DMA,
        ],
    )
    def k(codes_hbm, lut_hbm, out_hbm, lut_v, in_v0, in_v1, out_v0, out_v1,
          isem0, isem1, osem0, osem1):
        wid = lax.axis_index("s") * 2 + lax.axis_index("c")
        base_row = wid * rows_per_w
        pltpu.sync_copy(lut_hbm, lut_v)
        in_bufs = (in_v0, in_v1)
        out_bufs = (out_v0, out_v1)
        isems = (isem0, isem1)
        osems = (osem0, osem1)
        in_cps = [None, None]
        out_cps = [None, None]

        def start_in(g):
            b = g % 2
            in_cps[b] = pltpu.async_copy(
                codes_hbm.at[pl.ds(base_row + g * _ROWS_PER_CHUNK,
                                   _ROWS_PER_CHUNK), :],
                in_bufs[b], isems[b])

        start_in(0)
        for g in range(n_chunks):
            b = g % 2
            if g + 1 < n_chunks:
                start_in(g + 1)
            in_cps[b].wait()
            if out_cps[b] is not None:
                out_cps[b].wait()
            in_v, out_v = in_bufs[b], out_bufs[b]

            @plsc.parallel_loop(0, _ROWS_PER_CHUNK, 1, unroll=2)
            def body(p):
                for off in offs:
                    idx = in_v[p, pl.ds(off, _LANES)]
                    out_v[p, pl.ds(off, _LANES)] = plsc.load_gather(
                        lut_v, [idx])

            out_cps[b] = pltpu.async_copy(
                out_bufs[b],
                out_hbm.at[pl.ds(base_row + g * _ROWS_PER_CHUNK,
                                 _ROWS_PER_CHUNK), :], osems[b])

        for b in range(2):
            if out_cps[b] is not None:
                out_cps[b].wait()

    return k(codes, lut32)


def kernel(char_bytes, lut):
    B, L = char_bytes.shape
    lut32 = lut.astype(jnp.int32)
    out = _lut_gather(B, L, char_bytes.astype(jnp.int32), lut32)
    return out.astype(lut.dtype)


# speedup vs baseline: 525.7838x; 1.0003x over previous
import functools

import jax
import jax.numpy as jnp
from jax import lax
from jax.experimental import pallas as pl
from jax.experimental.pallas import tpu as pltpu
from jax.experimental.pallas import tpu_sc as plsc

_NW = 32
_LANES = 16
_ROWS_PER_CHUNK = 64


@functools.partial(jax.jit, static_argnums=(0, 1))
def _lut_gather(n_rows, n_cols, codes, lut32):
    rows_per_w = n_rows // _NW
    n_chunks = rows_per_w // _ROWS_PER_CHUNK
    offs = list(range(0, n_cols - _LANES + 1, _LANES))
    if offs[-1] + _LANES < n_cols:
        offs.append(n_cols - _LANES)
    mesh = plsc.VectorSubcoreMesh(core_axis_name="c", subcore_axis_name="s")

    @functools.partial(
        pl.kernel,
        out_type=jax.ShapeDtypeStruct((n_rows, n_cols), jnp.int32),
        mesh=mesh,
        compiler_params=pltpu.CompilerParams(
            needs_layout_passes=False, use_tc_tiling_on_sc=True),
        scratch_types=[
            pltpu.VMEM((256,), jnp.int32),
            pltpu.VMEM((_ROWS_PER_CHUNK, n_cols), jnp.int32),
            pltpu.VMEM((_ROWS_PER_CHUNK, n_cols), jnp.int32),
            pltpu.VMEM((_ROWS_PER_CHUNK, n_cols), jnp.int32),
            pltpu.VMEM((_ROWS_PER_CHUNK, n_cols), jnp.int32),
            pltpu.SemaphoreType.DMA,
            pltpu.SemaphoreType.DMA,
            pltpu.SemaphoreType.DMA,
            pltpu.SemaphoreType.DMA,
        ],
    )
    def k(codes_hbm, lut_hbm, out_hbm, lut_v, in_v0, in_v1, out_v0, out_v1,
          isem0, isem1, osem0, osem1):
        wid = lax.axis_index("s") * 2 + lax.axis_index("c")
        base_row = wid * rows_per_w
        pltpu.sync_copy(lut_hbm, lut_v)
        in_bufs = (in_v0, in_v1)
        out_bufs = (out_v0, out_v1)
        isems = (isem0, isem1)
        osems = (osem0, osem1)
        in_cps = [None, None]
        out_cps = [None, None]

        def start_in(g):
            b = g % 2
            in_cps[b] = pltpu.async_copy(
                codes_hbm.at[pl.ds(base_row + g * _ROWS_PER_CHUNK,
                                   _ROWS_PER_CHUNK), :],
                in_bufs[b], isems[b])

        start_in(0)
        for g in range(n_chunks):
            b = g % 2
            if g + 1 < n_chunks:
                start_in(g + 1)
            in_cps[b].wait()
            if out_cps[b] is not None:
                out_cps[b].wait()
            in_v, out_v = in_bufs[b], out_bufs[b]

            @plsc.parallel_loop(0, _ROWS_PER_CHUNK, 1, unroll=2)
            def body(p):
                for off in offs:
                    idx = in_v[p, pl.ds(off, _LANES)]
                    out_v[p, pl.ds(off, _LANES)] = plsc.load_gather(
                        lut_v, [idx])

            out_cps[b] = pltpu.async_copy(
                out_bufs[b],
                out_hbm.at[pl.ds(base_row + g * _ROWS_PER_CHUNK,
                                 _ROWS_PER_CHUNK), :], osems[b])

        for b in range(2):
            if out_cps[b] is not None:
                out_cps[b].wait()

    return k(codes, lut32)


def kernel(char_bytes, lut):
    B, L = char_bytes.shape
    lut32 = lut.astype(jnp.int32)
    out = _lut_gather(B, L, char_bytes.astype(jnp.int32), lut32)
    return out.astype(lut.dtype)


# speedup vs baseline: 969.9377x; 1.8447x over previous
import functools

import jax
import jax.numpy as jnp
from jax import lax
from jax.experimental import pallas as pl
from jax.experimental.pallas import tpu as pltpu
from jax.experimental.pallas import tpu_sc as plsc

_NW = 32
_LANES = 16
_COLS_PER_CHUNK = 128


@functools.partial(jax.jit, static_argnums=(0, 1))
def _lut_gather(n_rows, n_cols, codes, lut32):
    cols_per_w = n_cols // _NW
    n_chunks = cols_per_w // _COLS_PER_CHUNK
    n_j = _COLS_PER_CHUNK // _LANES
    mesh = plsc.VectorSubcoreMesh(core_axis_name="c", subcore_axis_name="s")

    @functools.partial(
        pl.kernel,
        out_type=jax.ShapeDtypeStruct((n_rows, n_cols), jnp.int32),
        mesh=mesh,
        compiler_params=pltpu.CompilerParams(
            needs_layout_passes=False, use_tc_tiling_on_sc=True),
        scratch_types=[
            pltpu.VMEM((256,), jnp.int32),
            pltpu.VMEM((n_rows, _COLS_PER_CHUNK), jnp.int32),
            pltpu.VMEM((n_rows, _COLS_PER_CHUNK), jnp.int32),
            pltpu.VMEM((n_rows, _COLS_PER_CHUNK), jnp.int32),
            pltpu.VMEM((n_rows, _COLS_PER_CHUNK), jnp.int32),
            pltpu.SemaphoreType.DMA,
            pltpu.SemaphoreType.DMA,
            pltpu.SemaphoreType.DMA,
            pltpu.SemaphoreType.DMA,
        ],
    )
    def k(codes_hbm, lut_hbm, out_hbm, lut_v, in_v0, in_v1, out_v0, out_v1,
          isem0, isem1, osem0, osem1):
        wid = lax.axis_index("s") * 2 + lax.axis_index("c")
        base_col = wid * cols_per_w
        pltpu.sync_copy(lut_hbm, lut_v)
        in_bufs = (in_v0, in_v1)
        out_bufs = (out_v0, out_v1)
        isems = (isem0, isem1)
        osems = (osem0, osem1)
        in_cps = [None, None]
        out_cps = [None, None]

        def start_in(g):
            b = g % 2
            in_cps[b] = pltpu.async_copy(
                codes_hbm.at[:, pl.ds(base_col + g * _COLS_PER_CHUNK,
                                      _COLS_PER_CHUNK)],
                in_bufs[b], isems[b])

        start_in(0)
        for g in range(n_chunks):
            b = g % 2
            if g + 1 < n_chunks:
                start_in(g + 1)
            in_cps[b].wait()
            if out_cps[b] is not None:
                out_cps[b].wait()
            in_v, out_v = in_bufs[b], out_bufs[b]

            @plsc.parallel_loop(0, n_rows, 1, unroll=2)
            def body(p):
                for j in range(n_j):
                    idx = in_v[p, pl.ds(j * _LANES, _LANES)]
                    out_v[p, pl.ds(j * _LANES, _LANES)] = plsc.load_gather(
                        lut_v, [idx])

            out_cps[b] = pltpu.async_copy(
                out_bufs[b],
                out_hbm.at[:, pl.ds(base_col + g * _COLS_PER_CHUNK,
                                    _COLS_PER_CHUNK)], osems[b])

        for b in range(2):
            if out_cps[b] is not None:
                out_cps[b].wait()

    return k(codes, lut32)


def kernel(char_bytes, lut):
    B, L = char_bytes.shape
    lut32 = lut.astype(jnp.int32)
    codes_t = char_bytes.astype(jnp.int32).T
    out_t = _lut_gather(L, B, codes_t, lut32)
    return out_t.T.astype(lut.dtype)
